# Initial kernel scaffold; baseline (speedup 1.0000x reference)
import jax
import jax.numpy as jnp
from jax.experimental import pallas as pl


def kernel(x, weights, indices, cdf):
    raise NotImplementedError("write your pallas kernel here")


# speedup vs baseline: 145.2294x; 145.2294x over previous
import functools

import jax
import jax.numpy as jnp
from jax import lax
from jax.experimental import pallas as pl
from jax.experimental.pallas import tpu as pltpu
from jax.experimental.pallas import tpu_sc as plsc

N = 65536
B = 16
M = 4096
K = 64
NC = 2
NS = 16
NW = NC * NS
ROWS = B * M
ROWS_PER_W = ROWS // NW
CHUNK = 128
NCHUNK = ROWS_PER_W // CHUNK
GROUPS = CHUNK // 16
L = 16


def _sc_weighted_gather(x, idx_flat, w_flat):
    mesh = plsc.VectorSubcoreMesh(core_axis_name="c", subcore_axis_name="s")

    @functools.partial(
        pl.kernel,
        out_type=jax.ShapeDtypeStruct((ROWS,), jnp.float32),
        mesh=mesh,
        compiler_params=pltpu.CompilerParams(needs_layout_passes=False),
        scratch_types=[
            pltpu.VMEM((N,), jnp.float32),
            pltpu.VMEM((CHUNK * K,), jnp.int32),
            pltpu.VMEM((CHUNK * K,), jnp.float32),
            pltpu.VMEM((ROWS_PER_W,), jnp.float32),
        ],
    )
    def k(x_hbm, idx_hbm, w_hbm, u_hbm, x_v, idx_v, w_v, u_v):
        wid = lax.axis_index("s") * NC + lax.axis_index("c")
        woff = wid * (ROWS_PER_W * K)
        pltpu.sync_copy(x_hbm, x_v)
        lane = lax.iota(jnp.int32, L)

        def chunk_body(c, _):
            base = woff + c * (CHUNK * K)
            pltpu.sync_copy(idx_hbm.at[pl.ds(base, CHUNK * K)], idx_v)
            pltpu.sync_copy(w_hbm.at[pl.ds(base, CHUNK * K)], w_v)

            def group_body(g, _):
                gbase = lane * K + g * (16 * K)
                acc0 = jnp.zeros((L,), jnp.float32)
                acc1 = jnp.zeros((L,), jnp.float32)
                acc2 = jnp.zeros((L,), jnp.float32)
                acc3 = jnp.zeros((L,), jnp.float32)
                accs = [acc0, acc1, acc2, acc3]
                for kk in range(K):
                    off = gbase + kk
                    iv = plsc.load_gather(idx_v, [off])
                    wv = plsc.load_gather(w_v, [off])
                    xg = plsc.load_gather(x_v, [iv])
                    accs[kk % 4] = accs[kk % 4] + xg * wv
                acc = (accs[0] + accs[1]) + (accs[2] + accs[3])
                u_v[pl.ds(c * CHUNK + g * 16, 16)] = acc
                return 0

            lax.fori_loop(0, GROUPS, group_body, 0)
            return 0

        lax.fori_loop(0, NCHUNK, chunk_body, 0)
        pltpu.sync_copy(u_v, u_hbm.at[pl.ds(wid * ROWS_PER_W, ROWS_PER_W)])

    return k(x, idx_flat, w_flat)


def _tc_loss_body(u_ref, cdf_ref, out_ref):
    u = u_ref[...]
    s = jnp.log1p(jnp.exp(u))
    t = (s[:, 1:] + s[:, :-1]) * 0.5
    trap = jnp.sum(t, axis=-1, keepdims=True)
    c = jnp.concatenate([jnp.zeros((B, 1), jnp.float32), t], axis=1)
    sh = 1
    while sh < M:
        c = c + jnp.concatenate(
            [jnp.zeros((B, sh), jnp.float32), c[:, :-sh]], axis=1)
        sh *= 2
    cdf_u = c[:, 1:] / trap
    d = cdf_ref[...] - cdf_u
    out_ref[0, 0] = jnp.sum(d * d) / (B * (M - 1))


def _tc_loss(u2d, cdf):
    out = pl.pallas_call(
        _tc_loss_body,
        out_shape=jax.ShapeDtypeStruct((1, 1), jnp.float32),
        out_specs=pl.BlockSpec(memory_space=pltpu.SMEM),
    )(u2d, cdf)
    return out[0, 0]


def kernel(x, weights, indices, cdf):
    idx_flat = indices.astype(jnp.int32).reshape(-1)
    w_flat = weights.reshape(-1)
    u = _sc_weighted_gather(x, idx_flat, w_flat)
    return _tc_loss(u.reshape(B, M), cdf)


# speedup vs baseline: 276.3641x; 1.9029x over previous
import functools

import jax
import jax.numpy as jnp
from jax import lax
from jax.experimental import pallas as pl
from jax.experimental.pallas import tpu as pltpu
from jax.experimental.pallas import tpu_sc as plsc

N = 65536
B = 16
M = 4096
K = 64
NC = 2
NS = 16
NW = NC * NS
ROWS = B * M
ROWS_PER_W = ROWS // NW
CHUNK = 128
NCHUNK = ROWS_PER_W // CHUNK
GROUPS = CHUNK // 16
CHUNK_WORDS = CHUNK * K
L = 16


def _sc_weighted_gather(x, idx_t, w_t):
    mesh = plsc.VectorSubcoreMesh(core_axis_name="c", subcore_axis_name="s")

    @functools.partial(
        pl.kernel,
        out_type=jax.ShapeDtypeStruct((ROWS,), jnp.float32),
        mesh=mesh,
        compiler_params=pltpu.CompilerParams(needs_layout_passes=False),
        scratch_types=[
            pltpu.VMEM((N,), jnp.float32),
            pltpu.VMEM((CHUNK_WORDS,), jnp.int32),
            pltpu.VMEM((CHUNK_WORDS,), jnp.int32),
            pltpu.VMEM((CHUNK_WORDS,), jnp.float32),
            pltpu.VMEM((CHUNK_WORDS,), jnp.float32),
            pltpu.VMEM((ROWS_PER_W,), jnp.float32),
            pltpu.SemaphoreType.DMA,
            pltpu.SemaphoreType.DMA,
        ],
    )
    def k(x_hbm, idx_hbm, w_hbm, u_hbm, x_v, idx_v0, idx_v1, w_v0, w_v1,
          u_v, sem_i, sem_w):
        idx_bufs = (idx_v0, idx_v1)
        w_bufs = (w_v0, w_v1)
        wid = lax.axis_index("s") * NC + lax.axis_index("c")
        woff = wid * (ROWS_PER_W * K)

        def dma_pair(c, slot):
            base = woff + c * CHUNK_WORDS
            hi = pltpu.make_async_copy(
                idx_hbm.at[pl.ds(base, CHUNK_WORDS)], idx_bufs[slot], sem_i)
            hw = pltpu.make_async_copy(
                w_hbm.at[pl.ds(base, CHUNK_WORDS)], w_bufs[slot], sem_w)
            return hi, hw

        hi0, hw0 = dma_pair(0, 0)
        hi0.start()
        hw0.start()
        pltpu.sync_copy(x_hbm, x_v)

        def compute_chunk(c, slot):
            idx_c = idx_bufs[slot]
            w_c = w_bufs[slot]

            def group_body(g, _):
                g16 = g * 16
                acc0 = jnp.zeros((L,), jnp.float32)
                acc1 = jnp.zeros((L,), jnp.float32)
                acc2 = jnp.zeros((L,), jnp.float32)
                acc3 = jnp.zeros((L,), jnp.float32)
                accs = [acc0, acc1, acc2, acc3]
                for kk in range(K):
                    iv = idx_c[pl.ds(kk * CHUNK + g16, 16)]
                    wv = w_c[pl.ds(kk * CHUNK + g16, 16)]
                    xg = plsc.load_gather(x_v, [iv])
                    accs[kk % 4] = accs[kk % 4] + xg * wv
                acc = (accs[0] + accs[1]) + (accs[2] + accs[3])
                u_v[pl.ds(c * CHUNK + g16, 16)] = acc
                return 0

            lax.fori_loop(0, GROUPS, group_body, 0)

        def pair_body(j, _):
            c0 = j * 2
            hi, hw = dma_pair(c0 + 1, 1)
            hi.start()
            hw.start()
            hi, hw = dma_pair(c0, 0)
            hi.wait()
            hw.wait()
            compute_chunk(c0, 0)

            @pl.when(c0 + 2 < NCHUNK)
            def _start_next():
                hi2, hw2 = dma_pair(c0 + 2, 0)
                hi2.start()
                hw2.start()

            hi, hw = dma_pair(c0 + 1, 1)
            hi.wait()
            hw.wait()
            compute_chunk(c0 + 1, 1)
            return 0

        lax.fori_loop(0, NCHUNK // 2, pair_body, 0)
        pltpu.sync_copy(u_v, u_hbm.at[pl.ds(wid * ROWS_PER_W, ROWS_PER_W)])

    return k(x, idx_t, w_t)


def _tc_loss_body(u_ref, cdf_ref, out_ref):
    u = u_ref[...]
    s = jnp.log1p(jnp.exp(u))
    t = (s[:, 1:] + s[:, :-1]) * 0.5
    trap = jnp.sum(t, axis=-1, keepdims=True)
    c = jnp.concatenate([jnp.zeros((B, 1), jnp.float32), t], axis=1)
    sh = 1
    while sh < M:
        c = c + jnp.concatenate(
            [jnp.zeros((B, sh), jnp.float32), c[:, :-sh]], axis=1)
        sh *= 2
    cdf_u = c[:, 1:] / trap
    d = cdf_ref[...] - cdf_u
    out_ref[0, 0] = jnp.sum(d * d) / (B * (M - 1))


def _tc_loss(u2d, cdf):
    out = pl.pallas_call(
        _tc_loss_body,
        out_shape=jax.ShapeDtypeStruct((1, 1), jnp.float32),
        out_specs=pl.BlockSpec(memory_space=pltpu.SMEM),
    )(u2d, cdf)
    return out[0, 0]


def kernel(x, weights, indices, cdf):
    idx_t = (indices.astype(jnp.int32)
             .reshape(NW, NCHUNK, CHUNK, K)
             .transpose(0, 1, 3, 2)
             .reshape(-1))
    w_t = (weights
           .reshape(NW, NCHUNK, CHUNK, K)
           .transpose(0, 1, 3, 2)
           .reshape(-1))
    u = _sc_weighted_gather(x, idx_t, w_t)
    return _tc_loss(u.reshape(B, M), cdf)


# speedup vs baseline: 291.7728x; 1.0558x over previous
import functools

import jax
import jax.numpy as jnp
from jax import lax
from jax.experimental import pallas as pl
from jax.experimental.pallas import tpu as pltpu
from jax.experimental.pallas import tpu_sc as plsc

N = 65536
B = 16
M = 4096
K = 64
NC = 2
NS = 16
NW = NC * NS
ROWS = B * M
ROWS_PER_W = ROWS // NW
CHUNK = 128
NCHUNK = ROWS_PER_W // CHUNK
GROUPS = CHUNK // 16
CHUNK_WORDS = CHUNK * K
L = 16


def _sc_weighted_gather(x, idx_t, w_t):
    mesh = plsc.VectorSubcoreMesh(core_axis_name="c", subcore_axis_name="s")

    @functools.partial(
        pl.kernel,
        out_type=jax.ShapeDtypeStruct((ROWS,), jnp.float32),
        mesh=mesh,
        compiler_params=pltpu.CompilerParams(needs_layout_passes=False),
        scratch_types=[
            pltpu.VMEM((N,), jnp.float32),
            pltpu.VMEM((CHUNK_WORDS,), jnp.int32),
            pltpu.VMEM((CHUNK_WORDS,), jnp.int32),
            pltpu.VMEM((CHUNK_WORDS,), jnp.float32),
            pltpu.VMEM((CHUNK_WORDS,), jnp.float32),
            pltpu.VMEM((ROWS_PER_W,), jnp.float32),
            pltpu.SemaphoreType.DMA,
            pltpu.SemaphoreType.DMA,
        ],
    )
    def k(x_hbm, idx_hbm, w_hbm, u_hbm, x_v, idx_v0, idx_v1, w_v0, w_v1,
          u_v, sem_i, sem_w):
        idx_bufs = (idx_v0, idx_v1)
        w_bufs = (w_v0, w_v1)
        wid = lax.axis_index("s") * NC + lax.axis_index("c")
        woff = wid * (ROWS_PER_W * K)

        def dma_pair(c, slot):
            base = woff + c * CHUNK_WORDS
            hi = pltpu.make_async_copy(
                idx_hbm.at[pl.ds(base, CHUNK_WORDS)], idx_bufs[slot], sem_i)
            hw = pltpu.make_async_copy(
                w_hbm.at[pl.ds(base, CHUNK_WORDS)], w_bufs[slot], sem_w)
            return hi, hw

        hi0, hw0 = dma_pair(0, 0)
        hi0.start()
        hw0.start()
        pltpu.sync_copy(x_hbm, x_v)

        def compute_chunk(c, slot):
            idx_c = idx_bufs[slot]
            w_c = w_bufs[slot]

            lane = lax.iota(jnp.int32, L)

            def group_body(g, _):
                res = jnp.zeros((L,), jnp.float32)
                for r16 in range(16):
                    base = (g * 16 + r16) * K
                    parts = []
                    for j in range(4):
                        iv = idx_c[pl.ds(base + j * 16, 16)]
                        wv = w_c[pl.ds(base + j * 16, 16)]
                        xg = plsc.load_gather(x_v, [iv])
                        parts.append(xg * wv)
                    acc = (parts[0] + parts[1]) + (parts[2] + parts[3])
                    res = jnp.where(lane == r16, jnp.sum(acc), res)
                u_v[pl.ds(c * CHUNK + g * 16, 16)] = res
                return 0

            lax.fori_loop(0, GROUPS, group_body, 0)

        def pair_body(j, _):
            c0 = j * 2
            hi, hw = dma_pair(c0 + 1, 1)
            hi.start()
            hw.start()
            hi, hw = dma_pair(c0, 0)
            hi.wait()
            hw.wait()
            compute_chunk(c0, 0)

            @pl.when(c0 + 2 < NCHUNK)
            def _start_next():
                hi2, hw2 = dma_pair(c0 + 2, 0)
                hi2.start()
                hw2.start()

            hi, hw = dma_pair(c0 + 1, 1)
            hi.wait()
            hw.wait()
            compute_chunk(c0 + 1, 1)
            return 0

        lax.fori_loop(0, NCHUNK // 2, pair_body, 0)
        pltpu.sync_copy(u_v, u_hbm.at[pl.ds(wid * ROWS_PER_W, ROWS_PER_W)])

    return k(x, idx_t, w_t)


def _tc_loss_body(u_ref, cdf_ref, out_ref):
    u = u_ref[...]
    s = jnp.log1p(jnp.exp(u))
    t = (s[:, 1:] + s[:, :-1]) * 0.5
    trap = jnp.sum(t, axis=-1, keepdims=True)
    c = jnp.concatenate([jnp.zeros((B, 1), jnp.float32), t], axis=1)
    sh = 1
    while sh < M:
        c = c + jnp.concatenate(
            [jnp.zeros((B, sh), jnp.float32), c[:, :-sh]], axis=1)
        sh *= 2
    cdf_u = c[:, 1:] / trap
    d = cdf_ref[...] - cdf_u
    out_ref[0, 0] = jnp.sum(d * d) / (B * (M - 1))


def _tc_loss(u2d, cdf):
    out = pl.pallas_call(
        _tc_loss_body,
        out_shape=jax.ShapeDtypeStruct((1, 1), jnp.float32),
        out_specs=pl.BlockSpec(memory_space=pltpu.SMEM),
    )(u2d, cdf)
    return out[0, 0]


def kernel(x, weights, indices, cdf):
    idx_flat = indices.astype(jnp.int32).reshape(-1)
    w_flat = weights.reshape(-1)
    u = _sc_weighted_gather(x, idx_flat, w_flat)
    return _tc_loss(u.reshape(B, M), cdf)


# speedup vs baseline: 328.5552x; 1.1261x over previous
import functools

import jax
import jax.numpy as jnp
from jax import lax
from jax.experimental import pallas as pl
from jax.experimental.pallas import tpu as pltpu
from jax.experimental.pallas import tpu_sc as plsc

N = 65536
B = 16
M = 4096
K = 64
NC = 2
NS = 16
NW = NC * NS
ROWS = B * M
ROWS_PER_W = ROWS // NW
CHUNK = 64
NCHUNK = ROWS_PER_W // CHUNK
GROUPS = CHUNK // 16
CHUNK_WORDS = CHUNK * K
L = 16


def _sc_weighted_gather(x, idx_t, w_t):
    mesh = plsc.VectorSubcoreMesh(core_axis_name="c", subcore_axis_name="s")

    @functools.partial(
        pl.kernel,
        out_type=jax.ShapeDtypeStruct((ROWS,), jnp.float32),
        mesh=mesh,
        compiler_params=pltpu.CompilerParams(needs_layout_passes=False),
        scratch_types=[
            pltpu.VMEM((N,), jnp.float32),
            pltpu.VMEM((CHUNK, K), jnp.int32),
            pltpu.VMEM((CHUNK, K), jnp.int32),
            pltpu.VMEM((CHUNK, K), jnp.float32),
            pltpu.VMEM((CHUNK, K), jnp.float32),
            pltpu.VMEM((ROWS_PER_W,), jnp.float32),
            pltpu.SemaphoreType.DMA,
            pltpu.SemaphoreType.DMA,
        ],
    )
    def k(x_hbm, idx_hbm, w_hbm, u_hbm, x_v, idx_v0, idx_v1, w_v0, w_v1,
          u_v, sem_i, sem_w):
        idx_bufs = (idx_v0, idx_v1)
        w_bufs = (w_v0, w_v1)
        wid = lax.axis_index("s") * NC + lax.axis_index("c")

        brow = wid * ROWS_PER_W // M
        m0 = (wid * ROWS_PER_W) % M

        def dma_pair(c, slot):
            ms = m0 + c * CHUNK
            hi = pltpu.make_async_copy(
                idx_hbm.at[brow, pl.ds(ms, CHUNK), :], idx_bufs[slot], sem_i)
            hw = pltpu.make_async_copy(
                w_hbm.at[brow, pl.ds(ms, CHUNK), :], w_bufs[slot], sem_w)
            return hi, hw

        hi0, hw0 = dma_pair(0, 0)
        hi0.start()
        hw0.start()
        pltpu.sync_copy(x_hbm, x_v)

        def compute_chunk(c, slot):
            idx_c = idx_bufs[slot]
            w_c = w_bufs[slot]

            lane = lax.iota(jnp.int32, L)

            def group_body(g, _):
                res = jnp.zeros((L,), jnp.float32)
                for r16 in range(16):
                    row = g * 16 + r16
                    parts = []
                    for j in range(4):
                        iv = idx_c[row, pl.ds(j * 16, 16)]
                        wv = w_c[row, pl.ds(j * 16, 16)]
                        xg = plsc.load_gather(x_v, [iv])
                        parts.append(xg * wv)
                    acc = (parts[0] + parts[1]) + (parts[2] + parts[3])
                    res = jnp.where(lane == r16, jnp.sum(acc), res)
                u_v[pl.ds(c * CHUNK + g * 16, 16)] = res
                return 0

            lax.fori_loop(0, GROUPS, group_body, 0)

        def pair_body(j, _):
            c0 = j * 2
            hi, hw = dma_pair(c0 + 1, 1)
            hi.start()
            hw.start()
            hi, hw = dma_pair(c0, 0)
            hi.wait()
            hw.wait()
            compute_chunk(c0, 0)

            @pl.when(c0 + 2 < NCHUNK)
            def _start_next():
                hi2, hw2 = dma_pair(c0 + 2, 0)
                hi2.start()
                hw2.start()

            hi, hw = dma_pair(c0 + 1, 1)
            hi.wait()
            hw.wait()
            compute_chunk(c0 + 1, 1)
            return 0

        lax.fori_loop(0, NCHUNK // 2, pair_body, 0)
        pltpu.sync_copy(u_v, u_hbm.at[pl.ds(wid * ROWS_PER_W, ROWS_PER_W)])

    return k(x, idx_t, w_t)


def _tc_loss_body(u_ref, cdf_ref, out_ref):
    u = u_ref[...]
    s = jnp.log1p(jnp.exp(u))
    t = (s[:, 1:] + s[:, :-1]) * 0.5
    trap = jnp.sum(t, axis=-1, keepdims=True)
    c = jnp.concatenate([jnp.zeros((B, 1), jnp.float32), t], axis=1)
    sh = 1
    while sh < M:
        c = c + jnp.concatenate(
            [jnp.zeros((B, sh), jnp.float32), c[:, :-sh]], axis=1)
        sh *= 2
    cdf_u = c[:, 1:] / trap
    d = cdf_ref[...] - cdf_u
    out_ref[0, 0] = jnp.sum(d * d) / (B * (M - 1))


def _tc_loss(u2d, cdf):
    out = pl.pallas_call(
        _tc_loss_body,
        out_shape=jax.ShapeDtypeStruct((1, 1), jnp.float32),
        out_specs=pl.BlockSpec(memory_space=pltpu.SMEM),
    )(u2d, cdf)
    return out[0, 0]


def kernel(x, weights, indices, cdf):
    u = _sc_weighted_gather(x, indices.astype(jnp.int32), weights)
    return _tc_loss(u.reshape(B, M), cdf)


# speedup vs baseline: 661.7157x; 2.0140x over previous
import functools

import jax
import jax.numpy as jnp
from jax import lax
from jax.experimental import pallas as pl
from jax.experimental.pallas import tpu as pltpu
from jax.experimental.pallas import tpu_sc as plsc

N = 65536
B = 16
M = 4096
K = 64
NC = 2
NS = 16
NW = NC * NS
ROWS = B * M
ROWS_PER_W = ROWS // NW
CHUNK = 128
NCHUNK = ROWS_PER_W // CHUNK
GROUPS = CHUNK // 16
L = 16


def _sc_weighted_gather(x, idx_t, w_t):
    mesh = plsc.VectorSubcoreMesh(core_axis_name="c", subcore_axis_name="s")

    @functools.partial(
        pl.kernel,
        out_type=jax.ShapeDtypeStruct((ROWS,), jnp.float32),
        mesh=mesh,
        compiler_params=pltpu.CompilerParams(needs_layout_passes=False),
        scratch_types=[
            pltpu.VMEM((N,), jnp.float32),
            pltpu.VMEM((K, CHUNK), jnp.int32),
            pltpu.VMEM((K, CHUNK), jnp.int32),
            pltpu.VMEM((K, CHUNK), jnp.float32),
            pltpu.VMEM((K, CHUNK), jnp.float32),
            pltpu.VMEM((ROWS_PER_W,), jnp.float32),
            pltpu.SemaphoreType.DMA,
            pltpu.SemaphoreType.DMA,
        ],
    )
    def k(x_hbm, idx_hbm, w_hbm, u_hbm, x_v, idx_v0, idx_v1, w_v0, w_v1,
          u_v, sem_i, sem_w):
        idx_bufs = (idx_v0, idx_v1)
        w_bufs = (w_v0, w_v1)
        wid = lax.axis_index("s") * NC + lax.axis_index("c")
        brow = wid * ROWS_PER_W // M
        m0 = (wid * ROWS_PER_W) % M

        def dma_pair(c, slot):
            ms = m0 + c * CHUNK
            hi = pltpu.make_async_copy(
                idx_hbm.at[brow, :, pl.ds(ms, CHUNK)], idx_bufs[slot], sem_i)
            hw = pltpu.make_async_copy(
                w_hbm.at[brow, :, pl.ds(ms, CHUNK)], w_bufs[slot], sem_w)
            return hi, hw

        hi0, hw0 = dma_pair(0, 0)
        hi0.start()
        hw0.start()
        pltpu.sync_copy(x_hbm, x_v)

        def compute_chunk(c, slot):
            idx_c = idx_bufs[slot]
            w_c = w_bufs[slot]

            def group_body(g, _):
                g16 = g * 16
                acc0 = jnp.zeros((L,), jnp.float32)
                acc1 = jnp.zeros((L,), jnp.float32)
                acc2 = jnp.zeros((L,), jnp.float32)
                acc3 = jnp.zeros((L,), jnp.float32)
                accs = [acc0, acc1, acc2, acc3]
                for kk in range(K):
                    iv = idx_c[kk, pl.ds(g16, 16)]
                    wv = w_c[kk, pl.ds(g16, 16)]
                    xg = plsc.load_gather(x_v, [iv])
                    accs[kk % 4] = accs[kk % 4] + xg * wv
                acc = (accs[0] + accs[1]) + (accs[2] + accs[3])
                u_v[pl.ds(c * CHUNK + g16, 16)] = acc
                return 0

            lax.fori_loop(0, GROUPS, group_body, 0)

        def pair_body(j, _):
            c0 = j * 2
            hi, hw = dma_pair(c0 + 1, 1)
            hi.start()
            hw.start()
            hi, hw = dma_pair(c0, 0)
            hi.wait()
            hw.wait()
            compute_chunk(c0, 0)

            @pl.when(c0 + 2 < NCHUNK)
            def _start_next():
                hi2, hw2 = dma_pair(c0 + 2, 0)
                hi2.start()
                hw2.start()

            hi, hw = dma_pair(c0 + 1, 1)
            hi.wait()
            hw.wait()
            compute_chunk(c0 + 1, 1)
            return 0

        lax.fori_loop(0, NCHUNK // 2, pair_body, 0)
        pltpu.sync_copy(u_v, u_hbm.at[pl.ds(wid * ROWS_PER_W, ROWS_PER_W)])

    return k(x, idx_t, w_t)


def _tc_loss_body(u_ref, cdf_ref, out_ref):
    u = u_ref[...]
    s = jnp.log1p(jnp.exp(u))
    t = (s[:, 1:] + s[:, :-1]) * 0.5
    trap = jnp.sum(t, axis=-1, keepdims=True)
    c = jnp.concatenate([jnp.zeros((B, 1), jnp.float32), t], axis=1)
    sh = 1
    while sh < M:
        c = c + jnp.concatenate(
            [jnp.zeros((B, sh), jnp.float32), c[:, :-sh]], axis=1)
        sh *= 2
    cdf_u = c[:, 1:] / trap
    d = cdf_ref[...] - cdf_u
    out_ref[0, 0] = jnp.sum(d * d) / (B * (M - 1))


def _tc_loss(u2d, cdf):
    out = pl.pallas_call(
        _tc_loss_body,
        out_shape=jax.ShapeDtypeStruct((1, 1), jnp.float32),
        out_specs=pl.BlockSpec(memory_space=pltpu.SMEM),
    )(u2d, cdf)
    return out[0, 0]


def kernel(x, weights, indices, cdf):
    idx_t = jnp.transpose(indices.astype(jnp.int32), (0, 2, 1))
    w_t = jnp.transpose(weights, (0, 2, 1))
    u = _sc_weighted_gather(x, idx_t, w_t)
    return _tc_loss(u.reshape(B, M), cdf)


# speedup vs baseline: 785.5539x; 1.1871x over previous
import functools

import jax
import jax.numpy as jnp
from jax import lax
from jax.experimental import pallas as pl
from jax.experimental.pallas import tpu as pltpu
from jax.experimental.pallas import tpu_sc as plsc

N = 65536
B = 16
M = 4096
K = 64
NC = 2
NS = 16
NW = NC * NS
ROWS = B * M
ROWS_PER_W = ROWS // NW
CHUNK = 128
NCHUNK = ROWS_PER_W // CHUNK
GROUPS = CHUNK // 16
L = 16
KU = 16


def _sc_weighted_gather(x, idx_t, w_t):
    mesh = plsc.VectorSubcoreMesh(core_axis_name="c", subcore_axis_name="s")

    @functools.partial(
        pl.kernel,
        out_type=jax.ShapeDtypeStruct((ROWS,), jnp.float32),
        mesh=mesh,
        compiler_params=pltpu.CompilerParams(needs_layout_passes=False),
        scratch_types=[
            pltpu.VMEM((N,), jnp.float32),
            pltpu.VMEM((K, CHUNK), jnp.int32),
            pltpu.VMEM((K, CHUNK), jnp.int32),
            pltpu.VMEM((K, CHUNK), jnp.float32),
            pltpu.VMEM((K, CHUNK), jnp.float32),
            pltpu.VMEM((ROWS_PER_W,), jnp.float32),
            pltpu.SemaphoreType.DMA,
            pltpu.SemaphoreType.DMA,
        ],
    )
    def k(x_hbm, idx_hbm, w_hbm, u_hbm, x_v, idx_v0, idx_v1, w_v0, w_v1,
          u_v, sem_i, sem_w):
        idx_bufs = (idx_v0, idx_v1)
        w_bufs = (w_v0, w_v1)
        wid = lax.axis_index("s") * NC + lax.axis_index("c")
        brow = wid * ROWS_PER_W // M
        m0 = (wid * ROWS_PER_W) % M

        def dma_pair(c, slot):
            ms = m0 + c * CHUNK
            hi = pltpu.make_async_copy(
                idx_hbm.at[brow, :, pl.ds(ms, CHUNK)], idx_bufs[slot], sem_i)
            hw = pltpu.make_async_copy(
                w_hbm.at[brow, :, pl.ds(ms, CHUNK)], w_bufs[slot], sem_w)
            return hi, hw

        hi0, hw0 = dma_pair(0, 0)
        hi0.start()
        hw0.start()
        pltpu.sync_copy(x_hbm, x_v)

        def compute_chunk(c, slot):
            idx_c = idx_bufs[slot]
            w_c = w_bufs[slot]

            def group_body(g, _):
                g16 = g * 16
                zero = jnp.zeros((L,), jnp.float32)

                def kblock(kb, accs):
                    accs = list(accs)
                    kbase = kb * KU
                    for t in range(KU):
                        iv = idx_c[kbase + t, pl.ds(g16, 16)]
                        wv = w_c[kbase + t, pl.ds(g16, 16)]
                        xg = plsc.load_gather(x_v, [iv])
                        accs[t % 4] = accs[t % 4] + xg * wv
                    return tuple(accs)

                accs = lax.fori_loop(
                    0, K // KU, kblock, (zero, zero, zero, zero))
                acc = (accs[0] + accs[1]) + (accs[2] + accs[3])
                u_v[pl.ds(c * CHUNK + g16, 16)] = acc
                return 0

            lax.fori_loop(0, GROUPS, group_body, 0)

        def pair_body(j, _):
            c0 = j * 2
            hi, hw = dma_pair(c0 + 1, 1)
            hi.start()
            hw.start()
            hi, hw = dma_pair(c0, 0)
            hi.wait()
            hw.wait()
            compute_chunk(c0, 0)

            @pl.when(c0 + 2 < NCHUNK)
            def _start_next():
                hi2, hw2 = dma_pair(c0 + 2, 0)
                hi2.start()
                hw2.start()

            hi, hw = dma_pair(c0 + 1, 1)
            hi.wait()
            hw.wait()
            compute_chunk(c0 + 1, 1)
            return 0

        lax.fori_loop(0, NCHUNK // 2, pair_body, 0)
        pltpu.sync_copy(u_v, u_hbm.at[pl.ds(wid * ROWS_PER_W, ROWS_PER_W)])

    return k(x, idx_t, w_t)


def _tc_loss_body(u_ref, cdf_ref, out_ref):
    u = u_ref[...]
    s = jnp.log1p(jnp.exp(u))
    t = (s[:, 1:] + s[:, :-1]) * 0.5
    trap = jnp.sum(t, axis=-1, keepdims=True)
    c = jnp.concatenate([jnp.zeros((B, 1), jnp.float32), t], axis=1)
    sh = 1
    while sh < M:
        c = c + jnp.concatenate(
            [jnp.zeros((B, sh), jnp.float32), c[:, :-sh]], axis=1)
        sh *= 2
    cdf_u = c[:, 1:] / trap
    d = cdf_ref[...] - cdf_u
    out_ref[0, 0] = jnp.sum(d * d) / (B * (M - 1))


def _tc_loss(u2d, cdf):
    out = pl.pallas_call(
        _tc_loss_body,
        out_shape=jax.ShapeDtypeStruct((1, 1), jnp.float32),
        out_specs=pl.BlockSpec(memory_space=pltpu.SMEM),
    )(u2d, cdf)
    return out[0, 0]


def kernel(x, weights, indices, cdf):
    idx_t = jnp.transpose(indices.astype(jnp.int32), (0, 2, 1))
    w_t = jnp.transpose(weights, (0, 2, 1))
    u = _sc_weighted_gather(x, idx_t, w_t)
    return _tc_loss(u.reshape(B, M), cdf)


# speedup vs baseline: 816.1826x; 1.0390x over previous
import functools

import jax
import jax.numpy as jnp
from jax import lax
from jax.experimental import pallas as pl
from jax.experimental.pallas import tpu as pltpu
from jax.experimental.pallas import tpu_sc as plsc

N = 65536
B = 16
M = 4096
K = 64
NC = 2
NS = 16
NW = NC * NS
ROWS = B * M
ROWS_PER_W = ROWS // NW
CHUNK = 128
NCHUNK = ROWS_PER_W // CHUNK
GROUPS = CHUNK // 16
L = 16
KU = 16


def _sc_weighted_gather(x, idx_t, w_t):
    mesh = plsc.VectorSubcoreMesh(core_axis_name="c", subcore_axis_name="s")

    @functools.partial(
        pl.kernel,
        out_type=jax.ShapeDtypeStruct((B, M), jnp.float32),
        mesh=mesh,
        compiler_params=pltpu.CompilerParams(needs_layout_passes=False),
        scratch_types=[
            pltpu.VMEM((N,), jnp.float32),
            pltpu.VMEM((K, CHUNK), jnp.int32),
            pltpu.VMEM((K, CHUNK), jnp.int32),
            pltpu.VMEM((K, CHUNK), jnp.float32),
            pltpu.VMEM((K, CHUNK), jnp.float32),
            pltpu.VMEM((ROWS_PER_W,), jnp.float32),
            pltpu.SemaphoreType.DMA,
            pltpu.SemaphoreType.DMA,
        ],
    )
    def k(x_hbm, idx_hbm, w_hbm, u_hbm, x_v, idx_v0, idx_v1, w_v0, w_v1,
          u_v, sem_i, sem_w):
        idx_bufs = (idx_v0, idx_v1)
        w_bufs = (w_v0, w_v1)
        wid = lax.axis_index("s") * NC + lax.axis_index("c")
        brow = wid * ROWS_PER_W // M
        m0 = (wid * ROWS_PER_W) % M

        def dma_pair(c, slot):
            ms = m0 + c * CHUNK
            hi = pltpu.make_async_copy(
                idx_hbm.at[brow, :, pl.ds(ms, CHUNK)], idx_bufs[slot], sem_i)
            hw = pltpu.make_async_copy(
                w_hbm.at[brow, :, pl.ds(ms, CHUNK)], w_bufs[slot], sem_w)
            return hi, hw

        hi0, hw0 = dma_pair(0, 0)
        hi0.start()
        hw0.start()
        pltpu.sync_copy(x_hbm, x_v)

        def compute_chunk(c, slot):
            idx_c = idx_bufs[slot]
            w_c = w_bufs[slot]

            def group_body(g, _):
                g16 = g * 16
                zero = jnp.zeros((L,), jnp.float32)

                def kblock(kb, accs):
                    accs = list(accs)
                    kbase = kb * KU
                    for t in range(KU):
                        iv = idx_c[kbase + t, pl.ds(g16, 16)]
                        wv = w_c[kbase + t, pl.ds(g16, 16)]
                        xg = plsc.load_gather(x_v, [iv])
                        accs[t % 4] = accs[t % 4] + xg * wv
                    return tuple(accs)

                accs = lax.fori_loop(
                    0, K // KU, kblock, (zero, zero, zero, zero))
                acc = (accs[0] + accs[1]) + (accs[2] + accs[3])
                u_v[pl.ds(c * CHUNK + g16, 16)] = acc
                return 0

            lax.fori_loop(0, GROUPS, group_body, 0)

        def pair_body(j, _):
            c0 = j * 2
            hi, hw = dma_pair(c0 + 1, 1)
            hi.start()
            hw.start()
            hi, hw = dma_pair(c0, 0)
            hi.wait()
            hw.wait()
            compute_chunk(c0, 0)

            @pl.when(c0 + 2 < NCHUNK)
            def _start_next():
                hi2, hw2 = dma_pair(c0 + 2, 0)
                hi2.start()
                hw2.start()

            hi, hw = dma_pair(c0 + 1, 1)
            hi.wait()
            hw.wait()
            compute_chunk(c0 + 1, 1)
            return 0

        lax.fori_loop(0, NCHUNK // 2, pair_body, 0)
        pltpu.sync_copy(u_v, u_hbm.at[brow, pl.ds(m0, ROWS_PER_W)])

    return k(x, idx_t, w_t)


def _tc_loss_body(u_ref, cdf_ref, out_ref):
    u = u_ref[...]
    s = jnp.log1p(jnp.exp(u))
    t = (s[:, 1:] + s[:, :-1]) * 0.5
    trap = jnp.sum(t, axis=-1, keepdims=True)
    c = jnp.concatenate([jnp.zeros((B, 1), jnp.float32), t], axis=1)
    sh = 1
    while sh < M:
        c = c + jnp.concatenate(
            [jnp.zeros((B, sh), jnp.float32), c[:, :-sh]], axis=1)
        sh *= 2
    cdf_u = c[:, 1:] / trap
    d = cdf_ref[...] - cdf_u
    out_ref[0, 0] = jnp.sum(d * d) / (B * (M - 1))


def _tc_loss(u2d, cdf):
    out = pl.pallas_call(
        _tc_loss_body,
        out_shape=jax.ShapeDtypeStruct((1, 1), jnp.float32),
        out_specs=pl.BlockSpec(memory_space=pltpu.SMEM),
    )(u2d, cdf)
    return out[0, 0]


def kernel(x, weights, indices, cdf):
    idx_t = jnp.transpose(indices.astype(jnp.int32), (0, 2, 1))
    w_t = jnp.transpose(weights, (0, 2, 1))
    u = _sc_weighted_gather(x, idx_t, w_t)
    return _tc_loss(u, cdf)
